# baseline (speedup 1.0000x reference)
import functools

import jax
import jax.numpy as jnp
from jax import lax
from jax.experimental import pallas as pl
from jax.experimental.pallas import tpu as pltpu
from jax.experimental.pallas import tpu_sc as plsc

_LANES = 16


def _build_lookup(B, V, D, num_workers, chunk, nbuf):
    b_per_w = B // num_workers
    n_chunks = b_per_w // chunk
    n_outer = n_chunks // nbuf
    assert n_outer * nbuf == n_chunks
    scale = float(D) ** 0.5
    mesh = plsc.VectorSubcoreMesh(core_axis_name="c", subcore_axis_name="s")
    nc = 2

    scratch = [pltpu.VMEM((b_per_w,), jnp.int32)]
    scratch += [pltpu.VMEM((chunk, D), jnp.float32) for _ in range(2 * nbuf)]
    scratch += [pltpu.SemaphoreType.DMA for _ in range(2 * nbuf)]

    @functools.partial(
        pl.kernel,
        mesh=mesh,
        out_type=jax.ShapeDtypeStruct((B, D), jnp.float32),
        scratch_types=scratch,
        compiler_params=pltpu.CompilerParams(use_tc_tiling_on_sc=False),
    )
    def lookup(idx_hbm, table_hbm, out_hbm, idx_v, *bufs):
        gb = bufs[:nbuf]
        sb = bufs[nbuf : 2 * nbuf]
        sem_g = bufs[2 * nbuf : 3 * nbuf]
        sem_s = bufs[3 * nbuf :]

        wid = lax.axis_index("s") * nc + lax.axis_index("c")
        base = wid * b_per_w
        pltpu.sync_copy(idx_hbm.at[pl.ds(base, b_per_w)], idx_v)

        def gather(off, b):
            return pltpu.make_async_copy(
                table_hbm.at[idx_v.at[pl.ds(off, chunk)]], gb[b], sem_g[b]
            )

        for b in range(nbuf):
            gather(b * chunk, b).start()

        def outer_body(o, carry):
            for b in range(nbuf):
                off = (o * nbuf + b) * chunk
                gather(off, b).wait()

                @pl.when(o > 0)
                def _wait_prev_store():
                    pltpu.make_async_copy(
                        sb[b], out_hbm.at[pl.ds(base, chunk)], sem_s[b]
                    ).wait()

                def row_body(r, c2):
                    for c in range(D // _LANES):
                        s = pl.ds(c * _LANES, _LANES)
                        sb[b][r, s] = gb[b][r, s] * scale
                    return c2

                lax.fori_loop(0, chunk, row_body, 0, unroll=4)
                pltpu.async_copy(
                    sb[b], out_hbm.at[pl.ds(base + off, chunk)], sem_s[b]
                )

                @pl.when(off + nbuf * chunk < b_per_w)
                def _next_gather():
                    gather(off + nbuf * chunk, b).start()

            return carry

        lax.fori_loop(0, n_outer, outer_body, 0)

        for b in range(nbuf):
            pltpu.make_async_copy(
                sb[b], out_hbm.at[pl.ds(base, chunk)], sem_s[b]
            ).wait()

    return lookup


def kernel(input_, table):
    B0, S = input_.shape
    V, D = table.shape
    B = B0 * S
    idx = input_.reshape(B).astype(jnp.int32)
    lookup = _build_lookup(B, V, D, num_workers=32, chunk=128, nbuf=4)
    out = lookup(idx, table)
    return out.reshape(B0, S, D)


# speedup vs baseline: 1.0405x; 1.0405x over previous
import functools

import jax
import jax.numpy as jnp
from jax import lax
from jax.experimental import pallas as pl
from jax.experimental.pallas import tpu as pltpu
from jax.experimental.pallas import tpu_sc as plsc

_LANES = 16


def _build_lookup(B0, S, D, num_workers, nbuf):
    rows_per_w = B0 // num_workers
    D2 = 2 * D
    c1 = 128
    c2 = S - c1
    scale = float(D) ** 0.5
    mesh = plsc.VectorSubcoreMesh(core_axis_name="c", subcore_axis_name="s")
    nc = 2

    scratch = [pltpu.VMEM((2, c1), jnp.int32) for _ in range(nbuf)]
    scratch += [pltpu.VMEM((S, D2), jnp.float32) for _ in range(nbuf)]
    scratch += [pltpu.VMEM((S, D), jnp.float32) for _ in range(nbuf)]
    scratch += [pltpu.SemaphoreType.DMA for _ in range(3 * nbuf)]

    @functools.partial(
        pl.kernel,
        mesh=mesh,
        out_type=jax.ShapeDtypeStruct((B0, S, D), jnp.float32),
        scratch_types=scratch,
        compiler_params=pltpu.CompilerParams(
            use_tc_tiling_on_sc=False, needs_layout_passes=False
        ),
    )
    def lookup(idx_hbm, table_hbm, out_hbm, *bufs):
        ib = bufs[:nbuf]
        gb = bufs[nbuf : 2 * nbuf]
        sb = bufs[2 * nbuf : 3 * nbuf]
        sem_i = bufs[3 * nbuf : 4 * nbuf]
        sem_g = bufs[4 * nbuf : 5 * nbuf]
        sem_s = bufs[5 * nbuf :]

        wid = lax.axis_index("s") * nc + lax.axis_index("c")
        row0 = wid * rows_per_w

        def idx_copies(g, b):
            off = (row0 + g) * S
            return (
                pltpu.make_async_copy(
                    idx_hbm.at[pl.ds(off, c1)], ib[b].at[0], sem_i[b]
                ),
                pltpu.make_async_copy(
                    idx_hbm.at[pl.ds(off + c1, c2)],
                    ib[b].at[1, pl.ds(0, c2)],
                    sem_i[b],
                ),
            )

        def gather_copies(b):
            return (
                pltpu.make_async_copy(
                    table_hbm.at[ib[b].at[0]], gb[b].at[pl.ds(0, c1)], sem_g[b]
                ),
                pltpu.make_async_copy(
                    table_hbm.at[ib[b].at[1, pl.ds(0, c2)]],
                    gb[b].at[pl.ds(c1, c2)],
                    sem_g[b],
                ),
            )

        def store_copy(g, b):
            return pltpu.make_async_copy(sb[b], out_hbm.at[row0 + g], sem_s[b])

        for cp in idx_copies(0, 0):
            cp.start()
        for cp in idx_copies(1, 1):
            cp.start()
        for cp in idx_copies(0, 0):
            cp.wait()
        for cp in gather_copies(0):
            cp.start()

        def visit(g, carry):
            b = lax.rem(g, nbuf)

            def on_buf(bs):
                @pl.when(b == bs)
                def _():
                    bn = (bs + 1) % nbuf
                    bn2 = (bs + 2) % nbuf

                    @pl.when(g + 1 < rows_per_w)
                    def _g1():
                        for cp in idx_copies(g + 1, bn):
                            cp.wait()
                        for cp in gather_copies(bn):
                            cp.start()

                    @pl.when(g + 2 < rows_per_w)
                    def _i2():
                        for cp in idx_copies(g + 2, bn2):
                            cp.start()

                    for cp in gather_copies(bs):
                        cp.wait()

                    @pl.when(g >= nbuf)
                    def _ws():
                        store_copy(0, bs).wait()

                    def row_body(r, c3):
                        for c in range(D // _LANES):
                            s = pl.ds(c * _LANES, _LANES)
                            sb[bs][r, s] = gb[bs][r, s] * scale
                        return c3

                    lax.fori_loop(0, S, row_body, 0, unroll=4)
                    store_copy(g, bs).start()

            for bs in range(nbuf):
                on_buf(bs)
            return carry

        lax.fori_loop(0, rows_per_w, visit, 0)

        for bs in range(nbuf):
            store_copy(0, bs).wait()

    return lookup


def kernel(input_, table):
    B0, S = input_.shape
    V, D = table.shape
    idx = input_.reshape(B0 * S).astype(jnp.int32)
    table_p = jnp.pad(table, ((0, 0), (0, D)))
    lookup = _build_lookup(B0, S, D, num_workers=32, nbuf=3)
    return lookup(idx, table_p)
